# baseline (speedup 1.0000x reference)
import functools

import jax
import jax.numpy as jnp
from jax import lax
from jax.experimental import pallas as pl
from jax.experimental.pallas import tpu as pltpu
from jax.experimental.pallas import tpu_sc as plsc

N = 10000
D = 128

NC = 2
NS = 16
NW = NC * NS

C = 128
ROWS_PER_SUB = 632
N_PAD = NS * ROWS_PER_SUB


def _agg_body(cpw0, cpw1, feat, srcr, dstr, zrow, agg_out,
              srcv0, srcv1, dstv0, dstv1, msgv0, msgv1,
              aggs, semg, sems0, sems1):
    c = lax.axis_index("c")
    s = lax.axis_index("s")
    pltpu.sync_copy(zrow, aggs.at[pl.ds(s * ROWS_PER_SUB, ROWS_PER_SUB)])
    plsc.subcore_barrier()

    cpw = jnp.where(c == 0, cpw0, cpw1)
    base = jnp.where(c == 0, s * cpw0, (NS * cpw0) + s * cpw1) * C

    srcv = (srcv0, srcv1)
    dstv = (dstv0, dstv1)
    msgv = (msgv0, msgv1)
    sems = (sems0, sems1)

    for b in range(2):
        off = base + b * C
        pltpu.sync_copy(srcr.at[pl.ds(off, C)], srcv[b])
        pltpu.sync_copy(dstr.at[pl.ds(off, C)], dstv[b])
        pltpu.async_copy(feat.at[srcv[b]], msgv[b], semg).wait()
        pltpu.async_copy(msgv[b], aggs.at[dstv[b]], sems[b], add=True)

    @pl.loop(0, cpw - 2, step=2)
    def _(g):
        for b in range(2):
            off = base + (g + 2 + b) * C
            pltpu.make_async_copy(msgv[b], aggs.at[dstv[b]],
                                  sems[b]).wait()
            pltpu.sync_copy(srcr.at[pl.ds(off, C)], srcv[b])
            pltpu.sync_copy(dstr.at[pl.ds(off, C)], dstv[b])
            pltpu.async_copy(feat.at[srcv[b]], msgv[b], semg).wait()
            pltpu.async_copy(msgv[b], aggs.at[dstv[b]], sems[b], add=True)

    for b in range(2):
        pltpu.make_async_copy(msgv[b], aggs.at[dstv[b]], sems[b]).wait()

    plsc.subcore_barrier()
    r0 = s * ROWS_PER_SUB
    pltpu.sync_copy(aggs.at[pl.ds(r0, ROWS_PER_SUB)],
                    agg_out.at[c, pl.ds(r0, ROWS_PER_SUB)])


def _make_agg(cpw0, cpw1):
    mesh = plsc.VectorSubcoreMesh(core_axis_name="c", subcore_axis_name="s")
    return pl.kernel(
        functools.partial(_agg_body, cpw0, cpw1),
        out_type=jax.ShapeDtypeStruct((NC, N_PAD, D), jnp.float32),
        mesh=mesh,
        scratch_types=[
            pltpu.VMEM((C,), jnp.int32),
            pltpu.VMEM((C,), jnp.int32),
            pltpu.VMEM((C,), jnp.int32),
            pltpu.VMEM((C,), jnp.int32),
            pltpu.VMEM((C, D), jnp.float32),
            pltpu.VMEM((C, D), jnp.float32),
            pltpu.VMEM_SHARED((N_PAD, D), jnp.float32),
            pltpu.SemaphoreType.DMA,
            pltpu.SemaphoreType.DMA,
            pltpu.SemaphoreType.DMA,
        ],
    )


def _cnt_body(cpw, dstr, zrow, ones_h, cnt_out, dstv, onesv, cnts):
    c = lax.axis_index("c")
    s = lax.axis_index("s")
    wid = c * NS + s
    pltpu.sync_copy(zrow, cnts.at[pl.ds(s * ROWS_PER_SUB, ROWS_PER_SUB)])
    pltpu.sync_copy(ones_h, onesv)
    plsc.subcore_barrier()

    base = wid * cpw * C

    @pl.loop(0, cpw)
    def _(i):
        pltpu.sync_copy(dstr.at[pl.ds(base + i * C, C)], dstv)
        pltpu.sync_copy(onesv, cnts.at[dstv], add=True)

    plsc.subcore_barrier()
    r0 = s * ROWS_PER_SUB
    pltpu.sync_copy(cnts.at[pl.ds(r0, ROWS_PER_SUB)],
                    cnt_out.at[c, pl.ds(r0, ROWS_PER_SUB)])


def _make_cnt(cpw):
    mesh = plsc.VectorSubcoreMesh(core_axis_name="c", subcore_axis_name="s")
    return pl.kernel(
        functools.partial(_cnt_body, cpw),
        out_type=jax.ShapeDtypeStruct((NC, N_PAD, D), jnp.float32),
        mesh=mesh,
        scratch_types=[
            pltpu.VMEM((C,), jnp.int32),
            pltpu.VMEM((C, D), jnp.float32),
            pltpu.VMEM_SHARED((N_PAD, D), jnp.float32),
        ],
    )


def _dense_body(relu, p0, p1, c0, c1, xr, wl, wr, b, out):
    cnt = jnp.clip(c0[...] + c1[...], 1.0, None)
    mean = (p0[...] + p1[...]) / cnt
    acc = jnp.dot(mean, wl[...], preferred_element_type=jnp.float32)
    acc = acc + jnp.dot(xr[...], wr[...], preferred_element_type=jnp.float32)
    acc = acc + b[...]
    if relu:
        acc = jnp.maximum(acc, 0.0)
    out[...] = acc


def _dense(p0, p1, c0, c1, x, wl, wr, b, relu):
    R = 1000
    grid = (N // R,)
    row_spec = pl.BlockSpec((R, D), lambda i: (i, 0))
    cnt_spec = pl.BlockSpec((R, 1), lambda i: (i, 0))
    w_spec = pl.BlockSpec((D, D), lambda i: (0, 0))
    b_spec = pl.BlockSpec((1, D), lambda i: (0, 0))
    return pl.pallas_call(
        functools.partial(_dense_body, relu),
        grid=grid,
        in_specs=[row_spec, row_spec, cnt_spec, cnt_spec, row_spec,
                  w_spec, w_spec, b_spec],
        out_specs=row_spec,
        out_shape=jax.ShapeDtypeStruct((N, D), jnp.float32),
    )(p0, p1, c0, c1, x, wl, wr, b)


def kernel(x, edge_index, W1_l, b1_l, W1_r, W2_l, b2_l, W2_r):
    E = edge_index.shape[1]
    frac0 = 0.80
    T = -(-E // C)
    A = max(32, int(round(T * frac0 / 32)) * 32)
    B = -(-(T - A) // 32) * 32
    e_pad = (A + B) * C
    src = edge_index[0].astype(jnp.int32)
    dst = edge_index[1].astype(jnp.int32)
    pad = e_pad - E
    if pad:
        src = jnp.concatenate([src, jnp.zeros((pad,), jnp.int32)])
        dst = jnp.concatenate([dst, jnp.full((pad,), N, jnp.int32)])

    zrow = jnp.zeros((ROWS_PER_SUB, D), jnp.float32)
    ones_h = jnp.ones((C, D), jnp.float32)

    agg = _make_agg(A // NS, B // NS)
    cntk = _make_cnt((A + B) // NW)
    b1 = b1_l.reshape(1, D)
    b2 = b2_l.reshape(1, D)

    cnt = cntk(dst, zrow, ones_h)
    c0 = cnt[0, :N, 0:1]
    c1 = cnt[1, :N, 0:1]
    a1 = agg(x, src, dst, zrow)
    h = _dense(a1[0, :N], a1[1, :N], c0, c1, x, W1_l, W1_r, b1, relu=True)
    a2 = agg(h, src, dst, zrow)
    out = _dense(a2[0, :N], a2[1, :N], c0, c1, h, W2_l, W2_r, b2, relu=False)
    return out


# speedup vs baseline: 1.0438x; 1.0438x over previous
import functools

import jax
import jax.numpy as jnp
from jax import lax
from jax.experimental import pallas as pl
from jax.experimental.pallas import tpu as pltpu
from jax.experimental.pallas import tpu_sc as plsc

N = 10000
D = 128

NC = 2
NS = 16
NW = NC * NS

C = 128
ROWS_PER_SUB = 632
N_PAD = NS * ROWS_PER_SUB


def _agg_body(cpw0, cpw1, feat, srcr, dstr, zrow, agg_out,
              srcv0, srcv1, dstv0, dstv1, msgv0, msgv1,
              aggs, semg, sems0, sems1):
    c = lax.axis_index("c")
    s = lax.axis_index("s")
    pltpu.sync_copy(zrow, aggs.at[pl.ds(s * ROWS_PER_SUB, ROWS_PER_SUB)])
    plsc.subcore_barrier()

    cpw = jnp.where(c == 0, cpw0, cpw1)
    base = jnp.where(c == 0, s * cpw0, (NS * cpw0) + s * cpw1) * C

    srcv = (srcv0, srcv1)
    dstv = (dstv0, dstv1)
    msgv = (msgv0, msgv1)
    sems = (sems0, sems1)

    for b in range(2):
        off = base + b * C
        pltpu.sync_copy(srcr.at[pl.ds(off, C)], srcv[b])
        pltpu.sync_copy(dstr.at[pl.ds(off, C)], dstv[b])
        pltpu.async_copy(feat.at[srcv[b]], msgv[b], semg).wait()
        pltpu.async_copy(msgv[b], aggs.at[dstv[b]], sems[b], add=True)

    @pl.loop(0, cpw - 2, step=2)
    def _(g):
        for b in range(2):
            off = base + (g + 2 + b) * C
            pltpu.make_async_copy(msgv[b], aggs.at[dstv[b]],
                                  sems[b]).wait()
            pltpu.sync_copy(srcr.at[pl.ds(off, C)], srcv[b])
            pltpu.sync_copy(dstr.at[pl.ds(off, C)], dstv[b])
            pltpu.async_copy(feat.at[srcv[b]], msgv[b], semg).wait()
            pltpu.async_copy(msgv[b], aggs.at[dstv[b]], sems[b], add=True)

    for b in range(2):
        pltpu.make_async_copy(msgv[b], aggs.at[dstv[b]], sems[b]).wait()

    plsc.subcore_barrier()
    r0 = s * ROWS_PER_SUB
    pltpu.sync_copy(aggs.at[pl.ds(r0, ROWS_PER_SUB)],
                    agg_out.at[c, pl.ds(r0, ROWS_PER_SUB)])


def _make_agg(cpw0, cpw1):
    mesh = plsc.VectorSubcoreMesh(core_axis_name="c", subcore_axis_name="s")
    return pl.kernel(
        functools.partial(_agg_body, cpw0, cpw1),
        out_type=jax.ShapeDtypeStruct((NC, N_PAD, D), jnp.float32),
        mesh=mesh,
        scratch_types=[
            pltpu.VMEM((C,), jnp.int32),
            pltpu.VMEM((C,), jnp.int32),
            pltpu.VMEM((C,), jnp.int32),
            pltpu.VMEM((C,), jnp.int32),
            pltpu.VMEM((C, D), jnp.float32),
            pltpu.VMEM((C, D), jnp.float32),
            pltpu.VMEM_SHARED((N_PAD, D), jnp.float32),
            pltpu.SemaphoreType.DMA,
            pltpu.SemaphoreType.DMA,
            pltpu.SemaphoreType.DMA,
        ],
    )


def _cnt_body(cpw, dstr, zrow, ones_h, cnt_out, dstv, onesv, cnts):
    c = lax.axis_index("c")
    s = lax.axis_index("s")
    wid = c * NS + s
    pltpu.sync_copy(zrow, cnts.at[pl.ds(s * ROWS_PER_SUB, ROWS_PER_SUB)])
    pltpu.sync_copy(ones_h, onesv)
    plsc.subcore_barrier()

    base = wid * cpw * C

    @pl.loop(0, cpw)
    def _(i):
        pltpu.sync_copy(dstr.at[pl.ds(base + i * C, C)], dstv)
        pltpu.sync_copy(onesv, cnts.at[dstv], add=True)

    plsc.subcore_barrier()
    r0 = s * ROWS_PER_SUB
    pltpu.sync_copy(cnts.at[pl.ds(r0, ROWS_PER_SUB)],
                    cnt_out.at[c, pl.ds(r0, ROWS_PER_SUB)])


def _make_cnt(cpw):
    mesh = plsc.VectorSubcoreMesh(core_axis_name="c", subcore_axis_name="s")
    return pl.kernel(
        functools.partial(_cnt_body, cpw),
        out_type=jax.ShapeDtypeStruct((NC, N_PAD, D), jnp.float32),
        mesh=mesh,
        scratch_types=[
            pltpu.VMEM((C,), jnp.int32),
            pltpu.VMEM((C, D), jnp.float32),
            pltpu.VMEM_SHARED((N_PAD, D), jnp.float32),
        ],
    )


def _dense_body(relu, p0, p1, c0, c1, xr, wl, wr, b, out):
    cnt = jnp.clip(c0[...] + c1[...], 1.0, None)
    mean = (p0[...] + p1[...]) / cnt
    acc = jnp.dot(mean, wl[...], preferred_element_type=jnp.float32)
    acc = acc + jnp.dot(xr[...], wr[...], preferred_element_type=jnp.float32)
    acc = acc + b[...]
    if relu:
        acc = jnp.maximum(acc, 0.0)
    out[...] = acc


def _dense(p0, p1, c0, c1, x, wl, wr, b, relu):
    R = 1000
    grid = (N // R,)
    row_spec = pl.BlockSpec((R, D), lambda i: (i, 0))
    cnt_spec = pl.BlockSpec((R, 1), lambda i: (i, 0))
    w_spec = pl.BlockSpec((D, D), lambda i: (0, 0))
    b_spec = pl.BlockSpec((1, D), lambda i: (0, 0))
    return pl.pallas_call(
        functools.partial(_dense_body, relu),
        grid=grid,
        in_specs=[row_spec, row_spec, cnt_spec, cnt_spec, row_spec,
                  w_spec, w_spec, b_spec],
        out_specs=row_spec,
        out_shape=jax.ShapeDtypeStruct((N, D), jnp.float32),
    )(p0, p1, c0, c1, x, wl, wr, b)


def kernel(x, edge_index, W1_l, b1_l, W1_r, W2_l, b2_l, W2_r):
    E = edge_index.shape[1]
    frac0 = 0.74
    T = -(-E // C)
    A = max(32, int(round(T * frac0 / 32)) * 32)
    B = -(-(T - A) // 32) * 32
    e_pad = (A + B) * C
    src = edge_index[0].astype(jnp.int32)
    dst = edge_index[1].astype(jnp.int32)
    pad = e_pad - E
    if pad:
        src = jnp.concatenate([src, jnp.zeros((pad,), jnp.int32)])
        dst = jnp.concatenate([dst, jnp.full((pad,), N, jnp.int32)])

    zrow = jnp.zeros((ROWS_PER_SUB, D), jnp.float32)
    ones_h = jnp.ones((C, D), jnp.float32)

    agg = _make_agg(A // NS, B // NS)
    cntk = _make_cnt((A + B) // NW)
    b1 = b1_l.reshape(1, D)
    b2 = b2_l.reshape(1, D)

    cnt = cntk(dst, zrow, ones_h)
    c0 = cnt[0, :N, 0:1]
    c1 = cnt[1, :N, 0:1]
    a1 = agg(x, src, dst, zrow)
    h = _dense(a1[0, :N], a1[1, :N], c0, c1, x, W1_l, W1_r, b1, relu=True)
    a2 = agg(h, src, dst, zrow)
    out = _dense(a2[0, :N], a2[1, :N], c0, c1, h, W2_l, W2_r, b2, relu=False)
    return out
